# baseline (speedup 1.0000x reference)
import jax
import jax.numpy as jnp
from jax.experimental import pallas as pl


def _copy_body(x_ref, o_ref):
    o_ref[...] = x_ref[...]


def kernel(x):
    n, c = x.shape
    m = n * c // 128
    xf = x.reshape(m, 128)
    br = 8192
    nb = m // br
    out = pl.pallas_call(
        _copy_body,
        grid=(nb,),
        in_specs=[pl.BlockSpec((br, 128), lambda i: ((i + nb // 2) % nb, 0))],
        out_specs=pl.BlockSpec((br, 128), lambda i: (i, 0)),
        out_shape=jax.ShapeDtypeStruct((m, 128), x.dtype),
    )(xf)
    return out.reshape(n, c)


# speedup vs baseline: 1.0895x; 1.0895x over previous
import functools

import jax
import jax.numpy as jnp
from jax import lax
from jax.experimental import pallas as pl
from jax.experimental.pallas import tpu as pltpu
from jax.experimental.pallas import tpu_sc as plsc

_NC = 2
_NS = 16
_CH = 256


@functools.partial(jax.jit, static_argnums=(1, 2))
def _sc_swap(x, n, c):
    nw = _NC * _NS
    rpw = n // nw
    half = n // 2
    nch = rpw // _CH
    mesh = plsc.VectorSubcoreMesh(core_axis_name="c", subcore_axis_name="s")

    @functools.partial(
        pl.kernel,
        mesh=mesh,
        out_type=jax.ShapeDtypeStruct((n, c), jnp.float32),
        scratch_types=[
            pltpu.VMEM((2, _CH, c), jnp.float32),
            pltpu.SemaphoreType.DMA,
            pltpu.SemaphoreType.DMA,
        ],
    )
    def k(x_hbm, out_hbm, buf, in_sem, out_sem):
        wid = lax.axis_index("s") * _NC + lax.axis_index("c")
        dst0 = wid * rpw
        src0 = lax.rem(dst0 + half, n)

        def in_copy(i, slot):
            return pltpu.make_async_copy(
                x_hbm.at[pl.ds(src0 + i * _CH, _CH)], buf.at[slot], in_sem)

        def out_copy(i, slot):
            return pltpu.make_async_copy(
                buf.at[slot], out_hbm.at[pl.ds(dst0 + i * _CH, _CH)], out_sem)

        in_copy(0, 0).start()

        def body(i, carry):
            slot = lax.rem(i, 2)
            nslot = lax.rem(i + 1, 2)

            @pl.when(i >= 1)
            def _():
                out_copy(i - 1, nslot).wait()

            @pl.when(i + 1 < nch)
            def _():
                in_copy(i + 1, nslot).start()

            in_copy(i, slot).wait()
            out_copy(i, slot).start()
            return carry

        lax.fori_loop(0, nch, body, 0)
        out_copy(nch - 1, lax.rem(nch - 1, 2)).wait()

    return k(x)


def kernel(x):
    n, c = x.shape
    return _sc_swap(x, n, c)
